# Initial kernel scaffold; baseline (speedup 1.0000x reference)
import jax
import jax.numpy as jnp
from jax.experimental import pallas as pl


def kernel(depth_pred, depth_gt, indices):
    raise NotImplementedError("write your pallas kernel here")


# speedup vs baseline: 1.3880x; 1.3880x over previous
import functools

import jax
import jax.numpy as jnp
from jax import lax
from jax.experimental import pallas as pl
from jax.experimental.pallas import tpu as pltpu
from jax.experimental.pallas import tpu_sc as plsc

DELTA = 0.1
SAMPLE_SIZE = 2500
B, H, W = 16, 512, 512
HW = H * W

PAD = 2560
HALF = PAD // 2
CHUNK = 128
NCHUNK = HALF // CHUNK
NVEC = CHUNK // 16


def _sc_body(dpf, dgf, i0_hbm, i1_hbm, out,
             i0_v, i1_v, p0_v, p1_v, g0_v, g1_v, acc_v, sem):
    s = lax.axis_index("s")
    c = lax.axis_index("c")
    b = s
    h = c
    wid = s * 2 + c

    pltpu.sync_copy(i0_hbm.at[h], i0_v)
    pltpu.sync_copy(i1_hbm.at[h], i1_v)

    off = (b * HW).astype(jnp.int32)
    for j in range(NCHUNK):
        for k in range(NVEC):
            sl = pl.ds(k * 16, 16)
            i0_v[j, sl] = i0_v[j, sl] + off
            i1_v[j, sl] = i1_v[j, sl] + off

    copies = []
    for j in range(NCHUNK):
        copies.append(pltpu.async_copy(dpf.at[i0_v.at[j]], p0_v.at[j], sem))
        copies.append(pltpu.async_copy(dpf.at[i1_v.at[j]], p1_v.at[j], sem))
        copies.append(pltpu.async_copy(dgf.at[i0_v.at[j]], g0_v.at[j], sem))
        copies.append(pltpu.async_copy(dgf.at[i1_v.at[j]], g1_v.at[j], sem))
    for cp in copies:
        cp.wait()

    lanes = lax.iota(jnp.int32, 16)
    base = h * HALF
    acc = jnp.zeros((16,), jnp.float32)
    for j in range(NCHUNK):
        for k in range(NVEC):
            sl = pl.ds(k * 16, 16)
            d = p0_v[j, sl] - p1_v[j, sl]
            g = g0_v[j, sl] - g1_v[j, sl]
            sq = d * d
            hinge = jnp.maximum(-d * jnp.sign(g), 0.0)
            loss = jnp.where(jnp.abs(g) < DELTA, sq, hinge)
            gid = base + (j * CHUNK + k * 16) + lanes
            acc = acc + jnp.where(gid < SAMPLE_SIZE, loss, 0.0)

    acc_v[...] = acc
    pltpu.sync_copy(acc_v, out.at[wid])


def _make_sc_kernel():
    mesh = plsc.VectorSubcoreMesh(core_axis_name="c", subcore_axis_name="s")
    return pl.kernel(
        _sc_body,
        out_type=jax.ShapeDtypeStruct((32, 16), jnp.float32),
        mesh=mesh,
        scratch_types=[
            pltpu.VMEM((NCHUNK, CHUNK), jnp.int32),
            pltpu.VMEM((NCHUNK, CHUNK), jnp.int32),
            pltpu.VMEM((NCHUNK, CHUNK), jnp.float32),
            pltpu.VMEM((NCHUNK, CHUNK), jnp.float32),
            pltpu.VMEM((NCHUNK, CHUNK), jnp.float32),
            pltpu.VMEM((NCHUNK, CHUNK), jnp.float32),
            pltpu.VMEM((16,), jnp.float32),
            pltpu.SemaphoreType.DMA,
        ],
    )


def _reduce_body(x_ref, o_ref):
    o_ref[0, 0] = jnp.sum(x_ref[...]) * (1.0 / (SAMPLE_SIZE * B))


_reduce = pl.pallas_call(
    _reduce_body,
    out_shape=jax.ShapeDtypeStruct((1, 1), jnp.float32),
    out_specs=pl.BlockSpec(memory_space=pltpu.SMEM),
)


@jax.jit
def kernel(depth_pred, depth_gt, indices):
    dpf = depth_pred.reshape(B * HW)
    dgf = depth_gt.reshape(B * HW)
    i0 = jnp.pad(indices[:, 0], (0, PAD - SAMPLE_SIZE)).reshape(2, NCHUNK, CHUNK)
    i1 = jnp.pad(indices[:, 1], (0, PAD - SAMPLE_SIZE)).reshape(2, NCHUNK, CHUNK)
    partials = _make_sc_kernel()(dpf, dgf, i0, i1)
    return _reduce(partials)[0, 0]
